# baseline (speedup 1.0000x reference)
import functools

import numpy as np

import jax
import jax.numpy as jnp
from jax import lax
from jax.experimental import pallas as pl
from jax.experimental.pallas import tpu as pltpu
from jax.experimental.pallas import tpu_sc as plsc

B, N, E, D, DE = 8, 4096, 65536, 128, 16
DG, DOUT, DGOUT, H, A = 64, 128, 64, 256, 2
BN, BE = B * N, B * E

NC, NS, L = 2, 16, 16
GPC = B // NC
EPS = E // NS
CE = 64
NCHUNK = EPS // CE
RPS = N // NS


def _mm_kernel(x_ref, w_ref, o_ref):
    o_ref[...] = jnp.dot(x_ref[...], w_ref[...],
                         preferred_element_type=jnp.float32)


def _ew_pack_kernel(x_ref, w_ref, b_ref, o_ref):
    f = jnp.dot(x_ref[...], w_ref[...],
                preferred_element_type=jnp.float32) + b_ref[...]
    lo = lax.bitcast_convert_type(
        f[:, :DOUT // 2].astype(jnp.bfloat16), jnp.uint16).astype(jnp.uint32)
    hi = lax.bitcast_convert_type(
        f[:, DOUT // 2:].astype(jnp.bfloat16), jnp.uint16).astype(jnp.uint32)
    o_ref[...] = lax.bitcast_convert_type(lo | (hi << 16), jnp.int32)


def _node_pool_kernel(x_ref, a_ref, wx_ref, wa_ref, b_ref, o_ref):
    i = pl.program_id(1)
    emb = jnp.dot(x_ref[0], wx_ref[...], preferred_element_type=jnp.float32)
    emb += jnp.dot(a_ref[0], wa_ref[...],
                   preferred_element_type=jnp.float32)
    emb = jnp.maximum(emb + b_ref[...], 0.0)
    s = jnp.sum(emb, axis=0)[None, None]

    @pl.when(i == 0)
    def _():
        o_ref[...] = s

    @pl.when(i != 0)
    def _():
        o_ref[...] += s


def _head_kernel(p_ref, gf_ref, wg_ref, bg_ref, w1p_ref, w1g_ref, b1_ref,
                 w2_ref, b2_ref, o_ref):
    pooled = p_ref[...] * (1.0 / N)
    eg = jnp.dot(gf_ref[...], wg_ref[...],
                 preferred_element_type=jnp.float32) + bg_ref[...]
    h = jnp.dot(pooled, w1p_ref[...], preferred_element_type=jnp.float32)
    h += jnp.dot(eg, w1g_ref[...], preferred_element_type=jnp.float32)
    h = jnp.maximum(h + b1_ref[...], 0.0)
    o_ref[...] = jnp.dot(h, w2_ref[...],
                         preferred_element_type=jnp.float32) + b2_ref[...]


def _sc_edge_body(xw_hbm, ew_hbm, src_hbm, dst_hbm, agg_hbm,
                  src_v, dst_v, xw_v0, xw_v1, ew_v0, ew_v1, out_v0, out_v1,
                  agg_sh, sg0, sg1, se0, se1, ss0, ss1):
    cid = lax.axis_index("c")
    sid = lax.axis_index("s")
    xw_bufs = (xw_v0, xw_v1)
    ew_bufs = (ew_v0, ew_v1)
    out_bufs = (out_v0, out_v1)
    gsems = (sg0, sg1)
    esems = (se0, se1)
    ssems = (ss0, ss1)

    z = jnp.zeros((L,), jnp.float32)
    himask = jnp.full((L,), -65536, jnp.int32)

    for gi in range(GPC):
        g = cid * GPC + gi
        row = g * NS + sid
        pltpu.sync_copy(src_hbm.at[row], src_v)
        pltpu.sync_copy(dst_hbm.at[row], dst_v)

        goff = jnp.full((L,), g * N, jnp.int32)

        def orow(n, _):
            for d in range(CE // L):
                src_v[n, pl.ds(d * L, L)] = (
                    src_v[n, pl.ds(d * L, L)] + goff)
            return ()

        lax.fori_loop(0, NCHUNK, orow, ())

        def zrow(e, _):
            for d in range(DOUT // L):
                out_v0[e, pl.ds(d * L, L)] = z
            return ()

        lax.fori_loop(0, CE, zrow, ())
        for k in range(RPS // CE):
            pltpu.sync_copy(out_v0, agg_sh.at[pl.ds(sid * RPS + k * CE, CE)])
        plsc.subcore_barrier()

        ebase = g * E + sid * EPS
        pltpu.async_copy(xw_hbm.at[src_v.at[0]], xw_v0, sg0)
        pltpu.async_copy(ew_hbm.at[pl.ds(ebase, CE)], ew_v0, se0)

        def pair(i, _):
            for b in range(2):
                j = i * 2 + b
                nb = 1 - b

                @pl.when(j + 1 < NCHUNK)
                def _():
                    pltpu.async_copy(xw_hbm.at[src_v.at[j + 1]],
                                     xw_bufs[nb], gsems[nb])
                    pltpu.async_copy(ew_hbm.at[pl.ds(ebase + (j + 1) * CE, CE)],
                                     ew_bufs[nb], esems[nb])

                pltpu.make_async_copy(xw_hbm.at[src_v.at[j]],
                                      xw_bufs[b], gsems[b]).wait()
                pltpu.make_async_copy(ew_hbm.at[pl.ds(ebase, CE)],
                                      ew_bufs[b], esems[b]).wait()

                @pl.when(j >= 2)
                def _():
                    pltpu.make_async_copy(out_bufs[b],
                                          agg_sh.at[dst_v.at[j]],
                                          ssems[b]).wait()

                def edge(e, _):
                    for d in range(DOUT // (2 * L)):
                        vi = ew_bufs[b][e, pl.ds(d * L, L)]
                        clo = lax.bitcast_convert_type(
                            lax.shift_left(vi, 16), jnp.float32)
                        chi = lax.bitcast_convert_type(
                            lax.bitwise_and(vi, himask), jnp.float32)
                        a0 = xw_bufs[b][e, pl.ds(2 * d * L, L)]
                        a1 = xw_bufs[b][e, pl.ds((2 * d + 1) * L, L)]
                        out_bufs[b][e, pl.ds(2 * d * L, L)] = (
                            jnp.maximum(a0 + clo, 0.0))
                        out_bufs[b][e, pl.ds((2 * d + 1) * L, L)] = (
                            jnp.maximum(a1 + chi, 0.0))
                    return ()

                lax.fori_loop(0, CE, edge, ())
                pltpu.async_copy(out_bufs[b], agg_sh.at[dst_v.at[j]],
                                 ssems[b], add=True)
            return ()

        lax.fori_loop(0, NCHUNK // 2, pair, ())
        pltpu.make_async_copy(out_v0, agg_sh.at[dst_v.at[0]], ss0).wait()
        pltpu.make_async_copy(out_v1, agg_sh.at[dst_v.at[1]], ss1).wait()
        plsc.subcore_barrier()
        pltpu.sync_copy(agg_sh.at[pl.ds(sid * RPS, RPS)],
                        agg_hbm.at[pl.ds(g * N + sid * RPS, RPS)])
        plsc.subcore_barrier()


def _sc_edge_aggregate(xw, ew, src_g, dst_l):
    mesh = plsc.VectorSubcoreMesh(core_axis_name="c", subcore_axis_name="s")
    return pl.kernel(
        _sc_edge_body,
        out_type=jax.ShapeDtypeStruct((BN, DOUT), jnp.float32),
        mesh=mesh,
        scratch_types=(
            [pltpu.VMEM((NCHUNK, CE), jnp.int32),
             pltpu.VMEM((NCHUNK, CE), jnp.int32),
             pltpu.VMEM((CE, DOUT), jnp.float32),
             pltpu.VMEM((CE, DOUT), jnp.float32),
             pltpu.VMEM((CE, DOUT // 2), jnp.int32),
             pltpu.VMEM((CE, DOUT // 2), jnp.int32),
             pltpu.VMEM((CE, DOUT), jnp.float32),
             pltpu.VMEM((CE, DOUT), jnp.float32),
             pltpu.VMEM_SHARED((N, DOUT), jnp.float32)]
            + [pltpu.SemaphoreType.DMA] * 6
        ),
    )(xw, ew, src_g, dst_l)


def kernel(node_features, edge_features, graph_features, edges_src,
           edges_dst, W_msg, b_msg, W_node, b_node, W_g, b_g, W1, b1,
           W2, b2):
    x = node_features.reshape(BN, D)
    e = edge_features.reshape(BE, DE)

    src_g = edges_src.reshape(B * NS, NCHUNK, CE)
    dst_l = edges_dst.reshape(B * NS, NCHUNK, CE)

    Wx, We = W_msg[:D], W_msg[D:]
    Wnx, Wna = W_node[:D], W_node[D:]

    trueidx = np.empty(DOUT, np.int32)
    for d in range(DOUT // 32):
        for i in range(16):
            trueidx[16 * d + i] = 32 * d + i
            trueidx[64 + 16 * d + i] = 32 * d + 16 + i
    We = We[:, trueidx]
    b_ew = b_msg[trueidx]


    BLK = 2048
    xw = pl.pallas_call(
        _mm_kernel,
        grid=(BN // BLK,),
        in_specs=[pl.BlockSpec((BLK, D), lambda i: (i, 0)),
                  pl.BlockSpec((D, DOUT), lambda i: (0, 0))],
        out_specs=pl.BlockSpec((BLK, DOUT), lambda i: (i, 0)),
        out_shape=jax.ShapeDtypeStruct((BN, DOUT), jnp.float32),
    )(x, Wx)

    BLK2 = 4096
    ew = pl.pallas_call(
        _ew_pack_kernel,
        grid=(BE // BLK2,),
        in_specs=[pl.BlockSpec((BLK2, DE), lambda i: (i, 0)),
                  pl.BlockSpec((DE, DOUT), lambda i: (0, 0)),
                  pl.BlockSpec((1, DOUT), lambda i: (0, 0))],
        out_specs=pl.BlockSpec((BLK2, DOUT // 2), lambda i: (i, 0)),
        out_shape=jax.ShapeDtypeStruct((BE, DOUT // 2), jnp.int32),
    )(e, We, b_ew.reshape(1, DOUT))

    agg = _sc_edge_aggregate(xw, ew, src_g, dst_l)

    BLK3 = 1024
    x3 = x.reshape(B, N, D)
    a3 = agg.reshape(B, N, DOUT)
    pooled = pl.pallas_call(
        _node_pool_kernel,
        grid=(B, N // BLK3),
        in_specs=[pl.BlockSpec((1, BLK3, D), lambda b, i: (b, i, 0)),
                  pl.BlockSpec((1, BLK3, DOUT), lambda b, i: (b, i, 0)),
                  pl.BlockSpec((D, DOUT), lambda b, i: (0, 0)),
                  pl.BlockSpec((DOUT, DOUT), lambda b, i: (0, 0)),
                  pl.BlockSpec((1, DOUT), lambda b, i: (0, 0))],
        out_specs=pl.BlockSpec((1, 1, DOUT), lambda b, i: (b, 0, 0)),
        out_shape=jax.ShapeDtypeStruct((B, 1, DOUT), jnp.float32),
    )(x3, a3, Wnx, Wna, b_node.reshape(1, DOUT))
    pooled = pooled.reshape(B, DOUT)

    logits = pl.pallas_call(
        _head_kernel,
        in_specs=[pl.BlockSpec((B, DOUT), lambda: (0, 0)),
                  pl.BlockSpec((B, DG), lambda: (0, 0)),
                  pl.BlockSpec((DG, DGOUT), lambda: (0, 0)),
                  pl.BlockSpec((1, DGOUT), lambda: (0, 0)),
                  pl.BlockSpec((DOUT, H), lambda: (0, 0)),
                  pl.BlockSpec((DGOUT, H), lambda: (0, 0)),
                  pl.BlockSpec((1, H), lambda: (0, 0)),
                  pl.BlockSpec((H, A), lambda: (0, 0)),
                  pl.BlockSpec((1, A), lambda: (0, 0))],
        out_specs=pl.BlockSpec((B, A), lambda: (0, 0)),
        out_shape=jax.ShapeDtypeStruct((B, A), jnp.float32),
    )(pooled, graph_features, W_g, b_g.reshape(1, DGOUT),
      W1[:DOUT], W1[DOUT:], b1.reshape(1, H), W2, b2.reshape(1, A))

    return logits


# speedup vs baseline: 1.1226x; 1.1226x over previous
import functools

import numpy as np

import jax
import jax.numpy as jnp
from jax import lax
from jax.experimental import pallas as pl
from jax.experimental.pallas import tpu as pltpu
from jax.experimental.pallas import tpu_sc as plsc

B, N, E, D, DE = 8, 4096, 65536, 128, 16
DG, DOUT, DGOUT, H, A = 64, 128, 64, 256, 2
BN, BE = B * N, B * E

NC, NS, L = 2, 16, 16
BH = B // 2
GPC = BH // NC
EPS = E // NS
CE = 64
NCHUNK = EPS // CE
RPS = N // NS


def _mm_kernel(x_ref, w_ref, o_ref):
    o_ref[...] = jnp.dot(x_ref[...], w_ref[...],
                         preferred_element_type=jnp.float32)


def _ew_pack_kernel(x_ref, w_ref, b_ref, o_ref):
    f = jnp.dot(x_ref[...], w_ref[...],
                preferred_element_type=jnp.float32) + b_ref[...]
    lo = lax.bitcast_convert_type(
        f[:, :DOUT // 2].astype(jnp.bfloat16), jnp.uint16).astype(jnp.uint32)
    hi = lax.bitcast_convert_type(
        f[:, DOUT // 2:].astype(jnp.bfloat16), jnp.uint16).astype(jnp.uint32)
    o_ref[...] = lax.bitcast_convert_type(lo | (hi << 16), jnp.int32)


def _node_pool_kernel(x_ref, a_ref, wx_ref, wa_ref, b_ref, o_ref):
    i = pl.program_id(1)
    emb = jnp.dot(x_ref[0], wx_ref[...], preferred_element_type=jnp.float32)
    emb += jnp.dot(a_ref[0], wa_ref[...],
                   preferred_element_type=jnp.float32)
    emb = jnp.maximum(emb + b_ref[...], 0.0)
    s = jnp.sum(emb, axis=0)[None, None]

    @pl.when(i == 0)
    def _():
        o_ref[...] = s

    @pl.when(i != 0)
    def _():
        o_ref[...] += s


def _head_kernel(p_ref, gf_ref, wg_ref, bg_ref, w1p_ref, w1g_ref, b1_ref,
                 w2_ref, b2_ref, o_ref):
    pooled = p_ref[...] * (1.0 / N)
    eg = jnp.dot(gf_ref[...], wg_ref[...],
                 preferred_element_type=jnp.float32) + bg_ref[...]
    h = jnp.dot(pooled, w1p_ref[...], preferred_element_type=jnp.float32)
    h += jnp.dot(eg, w1g_ref[...], preferred_element_type=jnp.float32)
    h = jnp.maximum(h + b1_ref[...], 0.0)
    o_ref[...] = jnp.dot(h, w2_ref[...],
                         preferred_element_type=jnp.float32) + b2_ref[...]


def _sc_edge_body(g0, xw_hbm, ew_hbm, src_hbm, dst_hbm, agg_hbm,
                  src_v, dst_v, xw_v0, xw_v1, ew_v0, ew_v1, out_v0, out_v1,
                  agg_sh, sg0, sg1, se0, se1, ss0, ss1):
    cid = lax.axis_index("c")
    sid = lax.axis_index("s")
    xw_bufs = (xw_v0, xw_v1)
    ew_bufs = (ew_v0, ew_v1)
    out_bufs = (out_v0, out_v1)
    gsems = (sg0, sg1)
    esems = (se0, se1)
    ssems = (ss0, ss1)

    z = jnp.zeros((L,), jnp.float32)
    himask = jnp.full((L,), -65536, jnp.int32)

    for gi in range(GPC):
        g = cid * GPC + gi
        row = (g0 + g) * NS + sid
        pltpu.sync_copy(src_hbm.at[row], src_v)
        pltpu.sync_copy(dst_hbm.at[row], dst_v)

        goff = jnp.full((L,), (g0 + g) * N, jnp.int32)

        def orow(n, _):
            for d in range(CE // L):
                src_v[n, pl.ds(d * L, L)] = (
                    src_v[n, pl.ds(d * L, L)] + goff)
            return ()

        lax.fori_loop(0, NCHUNK, orow, ())

        def zrow(e, _):
            for d in range(DOUT // L):
                out_v0[e, pl.ds(d * L, L)] = z
            return ()

        lax.fori_loop(0, CE, zrow, ())
        for k in range(RPS // CE):
            pltpu.sync_copy(out_v0, agg_sh.at[pl.ds(sid * RPS + k * CE, CE)])
        plsc.subcore_barrier()

        ebase = g * E + sid * EPS
        pltpu.async_copy(xw_hbm.at[src_v.at[0]], xw_v0, sg0)
        pltpu.async_copy(ew_hbm.at[pl.ds(ebase, CE)], ew_v0, se0)

        def pair(i, _):
            for b in range(2):
                j = i * 2 + b
                nb = 1 - b

                @pl.when(j + 1 < NCHUNK)
                def _():
                    pltpu.async_copy(xw_hbm.at[src_v.at[j + 1]],
                                     xw_bufs[nb], gsems[nb])
                    pltpu.async_copy(ew_hbm.at[pl.ds(ebase + (j + 1) * CE, CE)],
                                     ew_bufs[nb], esems[nb])

                pltpu.make_async_copy(xw_hbm.at[src_v.at[j]],
                                      xw_bufs[b], gsems[b]).wait()
                pltpu.make_async_copy(ew_hbm.at[pl.ds(ebase, CE)],
                                      ew_bufs[b], esems[b]).wait()

                @pl.when(j >= 2)
                def _():
                    pltpu.make_async_copy(out_bufs[b],
                                          agg_sh.at[dst_v.at[j]],
                                          ssems[b]).wait()

                def edge(e, _):
                    for d in range(DOUT // (2 * L)):
                        vi = ew_bufs[b][e, pl.ds(d * L, L)]
                        clo = lax.bitcast_convert_type(
                            lax.shift_left(vi, 16), jnp.float32)
                        chi = lax.bitcast_convert_type(
                            lax.bitwise_and(vi, himask), jnp.float32)
                        a0 = xw_bufs[b][e, pl.ds(2 * d * L, L)]
                        a1 = xw_bufs[b][e, pl.ds((2 * d + 1) * L, L)]
                        out_bufs[b][e, pl.ds(2 * d * L, L)] = (
                            jnp.maximum(a0 + clo, 0.0))
                        out_bufs[b][e, pl.ds((2 * d + 1) * L, L)] = (
                            jnp.maximum(a1 + chi, 0.0))
                    return ()

                lax.fori_loop(0, CE, edge, ())
                pltpu.async_copy(out_bufs[b], agg_sh.at[dst_v.at[j]],
                                 ssems[b], add=True)
            return ()

        lax.fori_loop(0, NCHUNK // 2, pair, ())
        pltpu.make_async_copy(out_v0, agg_sh.at[dst_v.at[0]], ss0).wait()
        pltpu.make_async_copy(out_v1, agg_sh.at[dst_v.at[1]], ss1).wait()
        plsc.subcore_barrier()
        pltpu.sync_copy(agg_sh.at[pl.ds(sid * RPS, RPS)],
                        agg_hbm.at[pl.ds(g * N + sid * RPS, RPS)])
        plsc.subcore_barrier()


def _sc_edge_aggregate(xw, ew, src_g, dst_l, g0):
    mesh = plsc.VectorSubcoreMesh(core_axis_name="c", subcore_axis_name="s")
    return pl.kernel(
        functools.partial(_sc_edge_body, g0),
        out_type=jax.ShapeDtypeStruct((BH * N, DOUT), jnp.float32),
        mesh=mesh,
        scratch_types=(
            [pltpu.VMEM((NCHUNK, CE), jnp.int32),
             pltpu.VMEM((NCHUNK, CE), jnp.int32),
             pltpu.VMEM((CE, DOUT), jnp.float32),
             pltpu.VMEM((CE, DOUT), jnp.float32),
             pltpu.VMEM((CE, DOUT // 2), jnp.int32),
             pltpu.VMEM((CE, DOUT // 2), jnp.int32),
             pltpu.VMEM((CE, DOUT), jnp.float32),
             pltpu.VMEM((CE, DOUT), jnp.float32),
             pltpu.VMEM_SHARED((N, DOUT), jnp.float32)]
            + [pltpu.SemaphoreType.DMA] * 6
        ),
    )(xw, ew, src_g, dst_l)


def kernel(node_features, edge_features, graph_features, edges_src,
           edges_dst, W_msg, b_msg, W_node, b_node, W_g, b_g, W1, b1,
           W2, b2):
    x = node_features.reshape(BN, D)
    e = edge_features.reshape(BE, DE)

    src_g = edges_src.reshape(B * NS, NCHUNK, CE)
    dst_l = edges_dst.reshape(B * NS, NCHUNK, CE)

    Wx, We = W_msg[:D], W_msg[D:]
    Wnx, Wna = W_node[:D], W_node[D:]

    trueidx = np.empty(DOUT, np.int32)
    for d in range(DOUT // 32):
        for i in range(16):
            trueidx[16 * d + i] = 32 * d + i
            trueidx[64 + 16 * d + i] = 32 * d + 16 + i
    We = We[:, trueidx]
    b_ew = b_msg[trueidx]


    BLK = 2048
    xw = pl.pallas_call(
        _mm_kernel,
        grid=(BN // BLK,),
        in_specs=[pl.BlockSpec((BLK, D), lambda i: (i, 0)),
                  pl.BlockSpec((D, DOUT), lambda i: (0, 0))],
        out_specs=pl.BlockSpec((BLK, DOUT), lambda i: (i, 0)),
        out_shape=jax.ShapeDtypeStruct((BN, DOUT), jnp.float32),
    )(x, Wx)

    BLK2 = 4096
    BEH = BE // 2

    def ew_half(half):
        off = half * (BEH // BLK2)
        return pl.pallas_call(
            _ew_pack_kernel,
            grid=(BEH // BLK2,),
            in_specs=[pl.BlockSpec((BLK2, DE), lambda i: (i + off, 0)),
                      pl.BlockSpec((DE, DOUT), lambda i: (0, 0)),
                      pl.BlockSpec((1, DOUT), lambda i: (0, 0))],
            out_specs=pl.BlockSpec((BLK2, DOUT // 2), lambda i: (i, 0)),
            out_shape=jax.ShapeDtypeStruct((BEH, DOUT // 2), jnp.int32),
        )(e, We, b_ew.reshape(1, DOUT))

    BLK3 = 1024
    x3 = x.reshape(B, N, D)

    def pool_half(half, a3h):
        off = half * BH
        return pl.pallas_call(
            _node_pool_kernel,
            grid=(BH, N // BLK3),
            in_specs=[pl.BlockSpec((1, BLK3, D),
                                   lambda b, i: (b + off, i, 0)),
                      pl.BlockSpec((1, BLK3, DOUT), lambda b, i: (b, i, 0)),
                      pl.BlockSpec((D, DOUT), lambda b, i: (0, 0)),
                      pl.BlockSpec((DOUT, DOUT), lambda b, i: (0, 0)),
                      pl.BlockSpec((1, DOUT), lambda b, i: (0, 0))],
            out_specs=pl.BlockSpec((1, 1, DOUT), lambda b, i: (b, 0, 0)),
            out_shape=jax.ShapeDtypeStruct((BH, 1, DOUT), jnp.float32),
        )(x3, a3h, Wnx, Wna, b_node.reshape(1, DOUT))

    ew0 = ew_half(0)
    agg0 = _sc_edge_aggregate(xw, ew0, src_g, dst_l, 0)
    ew1 = ew_half(1)
    agg1 = _sc_edge_aggregate(xw, ew1, src_g, dst_l, BH)
    pooled0 = pool_half(0, agg0.reshape(BH, N, DOUT))
    pooled1 = pool_half(1, agg1.reshape(BH, N, DOUT))
    pooled = jnp.concatenate([pooled0, pooled1], axis=0).reshape(B, DOUT)

    logits = pl.pallas_call(
        _head_kernel,
        in_specs=[pl.BlockSpec((B, DOUT), lambda: (0, 0)),
                  pl.BlockSpec((B, DG), lambda: (0, 0)),
                  pl.BlockSpec((DG, DGOUT), lambda: (0, 0)),
                  pl.BlockSpec((1, DGOUT), lambda: (0, 0)),
                  pl.BlockSpec((DOUT, H), lambda: (0, 0)),
                  pl.BlockSpec((DGOUT, H), lambda: (0, 0)),
                  pl.BlockSpec((1, H), lambda: (0, 0)),
                  pl.BlockSpec((H, A), lambda: (0, 0)),
                  pl.BlockSpec((1, A), lambda: (0, 0))],
        out_specs=pl.BlockSpec((B, A), lambda: (0, 0)),
        out_shape=jax.ShapeDtypeStruct((B, A), jnp.float32),
    )(pooled, graph_features, W_g, b_g.reshape(1, DGOUT),
      W1[:DOUT], W1[DOUT:], b1.reshape(1, H), W2, b2.reshape(1, A))

    return logits
